# baseline (speedup 1.0000x reference)
import jax
import jax.numpy as jnp
from jax.experimental import pallas as pl

TOK_BLOCK = 1024


def _gating_kernel(x_ref, w_ref, b_ref, out_ref):
    xb = x_ref[...].astype(jnp.bfloat16)
    wb = w_ref[...]
    logits = jax.lax.dot_general(
        xb, wb, (((1,), (1,)), ((), ())),
        preferred_element_type=jnp.float32,
    )
    logits = logits + b_ref[...]
    m = jnp.max(logits, axis=-1, keepdims=True)
    e = jnp.exp(logits - m)
    out_ref[...] = e / jnp.sum(e, axis=-1, keepdims=True)


def kernel(x, W, b):
    tokens, dim = x.shape
    experts = W.shape[0]
    b2 = b.reshape(1, experts)
    wb = W.astype(jnp.bfloat16)
    return pl.pallas_call(
        _gating_kernel,
        grid=(tokens // TOK_BLOCK,),
        in_specs=[
            pl.BlockSpec((TOK_BLOCK, dim), lambda i: (i, 0)),
            pl.BlockSpec((experts, dim), lambda i: (0, 0)),
            pl.BlockSpec((1, experts), lambda i: (0, 0)),
        ],
        out_specs=pl.BlockSpec((TOK_BLOCK, experts), lambda i: (i, 0)),
        out_shape=jax.ShapeDtypeStruct((tokens, experts), jnp.float32),
    )(x, wb, b2)
